# baseline (speedup 1.0000x reference)
import jax
import jax.numpy as jnp
from jax import lax
from jax.experimental import pallas as pl
from jax.experimental.pallas import tpu as pltpu
from jax.experimental.pallas import tpu_sc as plsc

N = 10000
E = 320000
D = 128

NC = 2
NS = 16
NW = NC * NS
SUB = 128
ROWS_PER_TILE = 80
R0 = 120
R1 = 40
EPAD = NW * ROWS_PER_TILE * SUB
N_TAB = 10016
DUMMY = N
NPT = 624
DEGW = 16

_MESH = plsc.VectorSubcoreMesh(core_axis_name="c", subcore_axis_name="s")


def _conv_sc_body(h_hbm, src_hbm, dst_hbm, m_out,
                  src_v, dst_v, rows_v, m_sh, sem):
    c = lax.axis_index("c")
    s = lax.axis_index("s")
    t = c * NS + s

    def _zrow(r, _):
        for cc in range(D // 16):
            rows_v[r, pl.ds(cc * 16, 16)] = jnp.zeros((16,), jnp.float32)
        return 0

    lax.fori_loop(0, SUB, _zrow, 0)

    base = s * NPT
    for j in range(NPT // SUB):
        pltpu.sync_copy(rows_v, m_sh.at[pl.ds(base + j * SUB, SUB)])
    pltpu.sync_copy(rows_v.at[pl.ds(0, NPT % SUB)],
                    m_sh.at[pl.ds(base + (NPT // SUB) * SUB, NPT % SUB)])

    @pl.when(s == NS - 1)
    def _zero_tail():
        off = NS * NPT
        pltpu.sync_copy(rows_v.at[pl.ds(0, N_TAB - NS * NPT)],
                        m_sh.at[pl.ds(off, N_TAB - NS * NPT)])

    pltpu.sync_copy(src_hbm.at[pl.ds(t * R0, R0)], src_v)
    pltpu.sync_copy(dst_hbm.at[pl.ds(t * R0, R0)], dst_v)

    plsc.subcore_barrier()

    nrows = lax.select(c == 0, R0, R1)

    def _edge_chunk(k, _):
        pltpu.async_copy(h_hbm.at[src_v.at[k]], rows_v, sem).wait()
        pltpu.sync_copy(rows_v, m_sh.at[dst_v.at[k]], add=True)
        return 0

    lax.fori_loop(0, nrows, _edge_chunk, 0)

    plsc.subcore_barrier()

    pltpu.sync_copy(m_sh.at[pl.ds(base, NPT)], m_out.at[c, pl.ds(base, NPT)])

    @pl.when(s == NS - 1)
    def _wb_tail():
        off = NS * NPT
        pltpu.sync_copy(m_sh.at[pl.ds(off, N - NS * NPT)],
                        m_out.at[c, pl.ds(off, N - NS * NPT)])


_conv_sc = pl.kernel(
    _conv_sc_body,
    out_type=pltpu.HBM((NC, N, D), jnp.float32),
    mesh=_MESH,
    scratch_types=[
        pltpu.VMEM((R0, SUB), jnp.int32),
        pltpu.VMEM((R0, SUB), jnp.int32),
        pltpu.VMEM((SUB, D), jnp.float32),
        pltpu.VMEM_SHARED((N_TAB, D), jnp.float32),
        pltpu.SemaphoreType.DMA,
    ],
    name="conv_sc")


def _deg_sc_body(dst_hbm, deg_out, dst_v, ones_v, deg_sh):
    c = lax.axis_index("c")
    s = lax.axis_index("s")
    t = c * NS + s

    def _fill(val):
        def _row(r, _):
            for cc in range(D // 16):
                ones_v[r, pl.ds(cc * 16, 16)] = jnp.full((16,), val,
                                                         jnp.float32)
            return 0
        lax.fori_loop(0, SUB, _row, 0)

    _fill(0.0)
    base = s * NPT
    for j in range(NPT // SUB):
        pltpu.sync_copy(ones_v, deg_sh.at[pl.ds(base + j * SUB, SUB)])
    pltpu.sync_copy(ones_v.at[pl.ds(0, NPT % SUB)],
                    deg_sh.at[pl.ds(base + (NPT // SUB) * SUB, NPT % SUB)])

    @pl.when(s == NS - 1)
    def _zero_tail():
        off = NS * NPT
        pltpu.sync_copy(ones_v.at[pl.ds(0, N_TAB - NS * NPT)],
                        deg_sh.at[pl.ds(off, N_TAB - NS * NPT)])

    _fill(1.0)
    pltpu.sync_copy(dst_hbm.at[pl.ds(t * ROWS_PER_TILE, ROWS_PER_TILE)], dst_v)

    plsc.subcore_barrier()

    def _edge_chunk(k, _):
        pltpu.sync_copy(ones_v, deg_sh.at[dst_v.at[k]], add=True)
        return 0

    lax.fori_loop(0, ROWS_PER_TILE, _edge_chunk, 0)

    plsc.subcore_barrier()

    pltpu.sync_copy(deg_sh.at[pl.ds(base, NPT)],
                    deg_out.at[c, pl.ds(base, NPT)])

    @pl.when(s == NS - 1)
    def _wb_tail():
        off = NS * NPT
        pltpu.sync_copy(deg_sh.at[pl.ds(off, N - NS * NPT)],
                        deg_out.at[c, pl.ds(off, N - NS * NPT)])


_deg_sc = pl.kernel(
    _deg_sc_body,
    out_type=pltpu.HBM((NC, N, D), jnp.float32),
    mesh=_MESH,
    scratch_types=[
        pltpu.VMEM((ROWS_PER_TILE, SUB), jnp.int32),
        pltpu.VMEM((SUB, D), jnp.float32),
        pltpu.VMEM_SHARED((N_TAB, D), jnp.float32),
    ],
    name="deg_sc")


def _enc_body(x_ref, w0_ref, b0_ref, w1_ref, b1_ref, o_ref):
    h = jnp.maximum(
        jnp.dot(x_ref[...], w0_ref[...], preferred_element_type=jnp.float32)
        + b0_ref[...], 0.0)
    o_ref[...] = jnp.maximum(
        jnp.dot(h, w1_ref[...], preferred_element_type=jnp.float32)
        + b1_ref[...], 0.0)


def _scale_body(d_ref, o_ref):
    deg = d_ref[0, :, 0:1] + d_ref[1, :, 0:1]
    o_ref[...] = 1.0 / jnp.maximum(deg, 1.0)


def _upd_body(m_ref, s_ref, h_ref, w_ref, b_ref, o_ref):
    m = m_ref[0] + m_ref[1]
    o_ref[...] = jnp.maximum(
        jnp.dot(m * s_ref[...], w_ref[...], preferred_element_type=jnp.float32)
        + b_ref[...] + h_ref[...], 0.0)


def _head_body(h_ref, w0_ref, b0_ref, w1t_ref, b1_ref, o_ref):
    obj = jnp.mean(h_ref[...], axis=0, keepdims=True)
    z = jnp.maximum(
        jnp.dot(obj, w0_ref[...], preferred_element_type=jnp.float32)
        + b0_ref[...], 0.0)
    o_ref[...] = jnp.sum(z * w1t_ref[...], axis=1, keepdims=True) + b1_ref[...]


def _tc_call(body, out_shape):
    return pl.pallas_call(body, out_shape=out_shape)


def kernel(x, edge_index, enc_W0, enc_b0, enc_W1, enc_b1,
           conv_W0, conv_b0, conv_W1, conv_b1, conv_W2, conv_b2,
           pred_W0, pred_b0, pred_W1, pred_b1):
    pad = EPAD - E
    pad_dst = DUMMY + (jnp.arange(pad, dtype=jnp.int32) % (N_TAB - N))
    src2d = jnp.concatenate(
        [edge_index[0], jnp.zeros((pad,), jnp.int32)]).reshape(EPAD // SUB, SUB)
    dst2d = jnp.concatenate(
        [edge_index[1], pad_dst]).reshape(EPAD // SUB, SUB)

    nd = NS * (R0 - R1) * SUB
    dummy_src = jnp.zeros((NS, R0 - R1, SUB), jnp.int32)
    dummy_dst = (DUMMY + (jnp.arange(nd, dtype=jnp.int32) % (N_TAB - N))
                 ).reshape(NS, R0 - R1, SUB)
    src3 = jnp.concatenate([
        src2d[:NS * R0],
        jnp.concatenate([src2d[NS * R0:].reshape(NS, R1, SUB), dummy_src],
                        axis=1).reshape(NS * (R0 - R1) + NS * R1, SUB)])
    dst3 = jnp.concatenate([
        dst2d[:NS * R0],
        jnp.concatenate([dst2d[NS * R0:].reshape(NS, R1, SUB), dummy_dst],
                        axis=1).reshape(NS * (R0 - R1) + NS * R1, SUB)])

    h = _tc_call(_enc_body, jax.ShapeDtypeStruct((N, D), jnp.float32))(
        x, enc_W0, enc_b0.reshape(1, D), enc_W1, enc_b1.reshape(1, D))

    deg_parts = _deg_sc(dst2d)
    scale = _tc_call(_scale_body, jax.ShapeDtypeStruct((N, 1), jnp.float32))(
        deg_parts)

    Ws = jnp.stack([conv_W0, conv_W1, conv_W2])
    bs = jnp.stack([conv_b0.reshape(1, D), conv_b1.reshape(1, D),
                    conv_b2.reshape(1, D)])

    def _layer(h_carry, Wb):
        W, b = Wb
        m_parts = _conv_sc(h_carry, src3, dst3)
        h_next = _tc_call(_upd_body, jax.ShapeDtypeStruct((N, D), jnp.float32))(
            m_parts, scale, h_carry, W, b)
        return h_next, 0

    h, _ = lax.scan(_layer, h, (Ws, bs))

    out = _tc_call(_head_body, jax.ShapeDtypeStruct((1, 1), jnp.float32))(
        h, pred_W0, pred_b0.reshape(1, D), pred_W1.reshape(1, D),
        pred_b1.reshape(1, 1))
    return out.reshape(())


# speedup vs baseline: 1.2729x; 1.2729x over previous
import jax
import jax.numpy as jnp
from jax import lax
from jax.experimental import pallas as pl
from jax.experimental.pallas import tpu as pltpu
from jax.experimental.pallas import tpu_sc as plsc

N = 10000
E = 320000
D = 128

NC = 2
NS = 16
NW = NC * NS
SUB = 128
ROWS_PER_TILE = 80
R0 = 120
R1 = 40
SBR = 64
EPAD = NW * ROWS_PER_TILE * SUB
CROWS = NW * R0 + 32
N_TAB = 10016
DUMMY = N
NPT = 624
DEGW = 16

_MESH = plsc.VectorSubcoreMesh(core_axis_name="c", subcore_axis_name="s")


def _conv_sc_body(h_hbm, src_hbm, dst_hbm, m_out,
                  src_v, dst_v, rows_a, rows_b, m_sh, sem_a, sem_b):
    c = lax.axis_index("c")
    s = lax.axis_index("s")
    t = c * NS + s

    def _zrow(r, _):
        for cc in range(D // 16):
            rows_a[r, pl.ds(cc * 16, 16)] = jnp.zeros((16,), jnp.float32)
        return 0

    lax.fori_loop(0, SUB, _zrow, 0)

    base = s * NPT
    for j in range(NPT // SUB):
        pltpu.sync_copy(rows_a, m_sh.at[pl.ds(base + j * SUB, SUB)])
    pltpu.sync_copy(rows_a.at[pl.ds(0, NPT % SUB)],
                    m_sh.at[pl.ds(base + (NPT // SUB) * SUB, NPT % SUB)])

    @pl.when(s == NS - 1)
    def _zero_tail():
        off = NS * NPT
        pltpu.sync_copy(rows_a.at[pl.ds(0, N_TAB - NS * NPT)],
                        m_sh.at[pl.ds(off, N_TAB - NS * NPT)])

    plsc.subcore_barrier()

    nrows = lax.select(c == 0, R0, R1)

    for sb in range(2):
        sb_off = t * R0 + sb * SBR
        pltpu.sync_copy(src_hbm.at[pl.ds(sb_off, SBR)], src_v)
        pltpu.sync_copy(dst_hbm.at[pl.ds(sb_off, SBR)], dst_v)
        cnt = jnp.clip(nrows - sb * SBR, 0, SBR)

        @pl.when(cnt > 0)
        def _prime():
            pltpu.async_copy(h_hbm.at[src_v.at[0]], rows_a, sem_a)

        def _pair(i, _):
            k0 = 2 * i
            k1 = k0 + 1
            pltpu.async_copy(h_hbm.at[src_v.at[k1]], rows_b, sem_b)
            pltpu.make_async_copy(h_hbm.at[src_v.at[0]], rows_a, sem_a).wait()
            pltpu.sync_copy(rows_a, m_sh.at[dst_v.at[k0]], add=True)
            k2 = jnp.minimum(k0 + 2, cnt - 1)
            pltpu.async_copy(h_hbm.at[src_v.at[k2]], rows_a, sem_a)
            pltpu.make_async_copy(h_hbm.at[src_v.at[0]], rows_b, sem_b).wait()
            pltpu.sync_copy(rows_b, m_sh.at[dst_v.at[k1]], add=True)
            return 0

        lax.fori_loop(0, cnt // 2, _pair, 0)

        @pl.when(cnt > 0)
        def _drain():
            pltpu.make_async_copy(h_hbm.at[src_v.at[0]], rows_a, sem_a).wait()

    plsc.subcore_barrier()

    pltpu.sync_copy(m_sh.at[pl.ds(base, NPT)], m_out.at[c, pl.ds(base, NPT)])

    @pl.when(s == NS - 1)
    def _wb_tail():
        off = NS * NPT
        pltpu.sync_copy(m_sh.at[pl.ds(off, N - NS * NPT)],
                        m_out.at[c, pl.ds(off, N - NS * NPT)])


_conv_sc = pl.kernel(
    _conv_sc_body,
    out_type=pltpu.HBM((NC, N, D), jnp.float32),
    mesh=_MESH,
    scratch_types=[
        pltpu.VMEM((SBR, SUB), jnp.int32),
        pltpu.VMEM((SBR, SUB), jnp.int32),
        pltpu.VMEM((SUB, D), jnp.float32),
        pltpu.VMEM((SUB, D), jnp.float32),
        pltpu.VMEM_SHARED((N_TAB, D), jnp.float32),
        pltpu.SemaphoreType.DMA,
        pltpu.SemaphoreType.DMA,
    ],
    name="conv_sc")


def _deg_sc_body(dst_hbm, deg_out, dst_v, ones_v, deg_sh):
    c = lax.axis_index("c")
    s = lax.axis_index("s")
    t = c * NS + s

    def _fill(val):
        def _row(r, _):
            for cc in range(D // 16):
                ones_v[r, pl.ds(cc * 16, 16)] = jnp.full((16,), val,
                                                         jnp.float32)
            return 0
        lax.fori_loop(0, SUB, _row, 0)

    _fill(0.0)
    base = s * NPT
    for j in range(NPT // SUB):
        pltpu.sync_copy(ones_v, deg_sh.at[pl.ds(base + j * SUB, SUB)])
    pltpu.sync_copy(ones_v.at[pl.ds(0, NPT % SUB)],
                    deg_sh.at[pl.ds(base + (NPT // SUB) * SUB, NPT % SUB)])

    @pl.when(s == NS - 1)
    def _zero_tail():
        off = NS * NPT
        pltpu.sync_copy(ones_v.at[pl.ds(0, N_TAB - NS * NPT)],
                        deg_sh.at[pl.ds(off, N_TAB - NS * NPT)])

    _fill(1.0)
    pltpu.sync_copy(dst_hbm.at[pl.ds(t * ROWS_PER_TILE, ROWS_PER_TILE)], dst_v)

    plsc.subcore_barrier()

    def _edge_chunk(k, _):
        pltpu.sync_copy(ones_v, deg_sh.at[dst_v.at[k]], add=True)
        return 0

    lax.fori_loop(0, ROWS_PER_TILE, _edge_chunk, 0)

    plsc.subcore_barrier()

    pltpu.sync_copy(deg_sh.at[pl.ds(base, NPT)],
                    deg_out.at[c, pl.ds(base, NPT)])

    @pl.when(s == NS - 1)
    def _wb_tail():
        off = NS * NPT
        pltpu.sync_copy(deg_sh.at[pl.ds(off, N - NS * NPT)],
                        deg_out.at[c, pl.ds(off, N - NS * NPT)])


_deg_sc = pl.kernel(
    _deg_sc_body,
    out_type=pltpu.HBM((NC, N, D), jnp.float32),
    mesh=_MESH,
    scratch_types=[
        pltpu.VMEM((ROWS_PER_TILE, SUB), jnp.int32),
        pltpu.VMEM((SUB, D), jnp.float32),
        pltpu.VMEM_SHARED((N_TAB, D), jnp.float32),
    ],
    name="deg_sc")


def _enc_body(x_ref, w0_ref, b0_ref, w1_ref, b1_ref, o_ref):
    h = jnp.maximum(
        jnp.dot(x_ref[...], w0_ref[...], preferred_element_type=jnp.float32)
        + b0_ref[...], 0.0)
    o_ref[...] = jnp.maximum(
        jnp.dot(h, w1_ref[...], preferred_element_type=jnp.float32)
        + b1_ref[...], 0.0)


def _scale_body(d_ref, o_ref):
    deg = d_ref[0, :, 0:1] + d_ref[1, :, 0:1]
    o_ref[...] = 1.0 / jnp.maximum(deg, 1.0)


def _upd_body(m_ref, s_ref, h_ref, w_ref, b_ref, o_ref):
    m = m_ref[0] + m_ref[1]
    o_ref[...] = jnp.maximum(
        jnp.dot(m * s_ref[...], w_ref[...], preferred_element_type=jnp.float32)
        + b_ref[...] + h_ref[...], 0.0)


def _head_body(h_ref, w0_ref, b0_ref, w1t_ref, b1_ref, o_ref):
    obj = jnp.mean(h_ref[...], axis=0, keepdims=True)
    z = jnp.maximum(
        jnp.dot(obj, w0_ref[...], preferred_element_type=jnp.float32)
        + b0_ref[...], 0.0)
    o_ref[...] = jnp.sum(z * w1t_ref[...], axis=1, keepdims=True) + b1_ref[...]


def _tc_call(body, out_shape):
    return pl.pallas_call(body, out_shape=out_shape)


def kernel(x, edge_index, enc_W0, enc_b0, enc_W1, enc_b1,
           conv_W0, conv_b0, conv_W1, conv_b1, conv_W2, conv_b2,
           pred_W0, pred_b0, pred_W1, pred_b1):
    pad = EPAD - E
    pad_dst = DUMMY + (jnp.arange(pad, dtype=jnp.int32) % (N_TAB - N))
    src2d = jnp.concatenate(
        [edge_index[0], jnp.zeros((pad,), jnp.int32)]).reshape(EPAD // SUB, SUB)
    dst2d = jnp.concatenate(
        [edge_index[1], pad_dst]).reshape(EPAD // SUB, SUB)

    nd = NS * (R0 - R1) * SUB
    dummy_src = jnp.zeros((NS, R0 - R1, SUB), jnp.int32)
    dummy_dst = (DUMMY + (jnp.arange(nd, dtype=jnp.int32) % (N_TAB - N))
                 ).reshape(NS, R0 - R1, SUB)
    tail = CROWS - NW * R0
    src3 = jnp.concatenate([
        src2d[:NS * R0],
        jnp.concatenate([src2d[NS * R0:].reshape(NS, R1, SUB), dummy_src],
                        axis=1).reshape(NS * (R0 - R1) + NS * R1, SUB),
        jnp.zeros((tail, SUB), jnp.int32)])
    dst3 = jnp.concatenate([
        dst2d[:NS * R0],
        jnp.concatenate([dst2d[NS * R0:].reshape(NS, R1, SUB), dummy_dst],
                        axis=1).reshape(NS * (R0 - R1) + NS * R1, SUB),
        jnp.full((tail, SUB), DUMMY, jnp.int32)])

    h = _tc_call(_enc_body, jax.ShapeDtypeStruct((N, D), jnp.float32))(
        x, enc_W0, enc_b0.reshape(1, D), enc_W1, enc_b1.reshape(1, D))

    deg_parts = _deg_sc(dst2d)
    scale = _tc_call(_scale_body, jax.ShapeDtypeStruct((N, 1), jnp.float32))(
        deg_parts)

    Ws = jnp.stack([conv_W0, conv_W1, conv_W2])
    bs = jnp.stack([conv_b0.reshape(1, D), conv_b1.reshape(1, D),
                    conv_b2.reshape(1, D)])

    def _layer(h_carry, Wb):
        W, b = Wb
        m_parts = _conv_sc(h_carry, src3, dst3)
        h_next = _tc_call(_upd_body, jax.ShapeDtypeStruct((N, D), jnp.float32))(
            m_parts, scale, h_carry, W, b)
        return h_next, 0

    h, _ = lax.scan(_layer, h, (Ws, bs))

    out = _tc_call(_head_body, jax.ShapeDtypeStruct((1, 1), jnp.float32))(
        h, pred_W0, pred_b0.reshape(1, D), pred_W1.reshape(1, D),
        pred_b1.reshape(1, 1))
    return out.reshape(())
